# baseline (speedup 1.0000x reference)
import jax
import jax.numpy as jnp
from jax import lax
from jax.experimental import pallas as pl
from jax.experimental.pallas import tpu as pltpu


def _body(x_ref, o_ref):
    x = x_ref[0]
    n = x.shape[0]
    s = jnp.sum(x * x, axis=1)
    m = jnp.max(s)
    iota = lax.broadcasted_iota(jnp.int32, (n,), 0)
    i = jnp.min(jnp.where(s == m, iota, n))
    o_ref[0, 0, :] = x_ref[0, pl.ds(i, 1), :][0]


def kernel(input):
    b, n, d = input.shape
    out = pl.pallas_call(
        _body,
        grid=(b,),
        in_specs=[pl.BlockSpec((1, n, d), lambda i: (i, 0, 0))],
        out_specs=pl.BlockSpec((1, 1, d), lambda i: (i, 0, 0)),
        out_shape=jax.ShapeDtypeStruct((b, 1, d), input.dtype),
    )(input)
    return out.reshape(b, d)


# speedup vs baseline: 1.3079x; 1.3079x over previous
import numpy as np
import jax
import jax.numpy as jnp
from jax import lax
from jax.experimental import pallas as pl


def _build_idmap():
    t = 2 * np.arange(4096)[:, None] + (np.arange(128)[None, :] >= 64)
    lane = np.arange(128)[None, :]
    w, r = 32, 4096
    while r > 64:
        mask = (lane % (2 * w)) < w
        t = np.where(mask, t[: r // 2], t[r // 2 :])
        r //= 2
        w //= 2
    return t.astype(np.int32)


_IDMAP = _build_idmap()


def _body(x_ref, idmap_ref, o_ref):
    lane = lax.broadcasted_iota(jnp.int32, (1, 128), 1)
    masks = {w: (lane % (2 * w)) < w for w in (32, 16, 8, 4, 2, 1)}
    y = x_ref[0]
    a, b = y[:2048], y[2048:]
    ar = jnp.roll(a, -32, axis=1)
    br = jnp.roll(b, 32, axis=1)
    t = jnp.where(masks[32], a * a + ar * ar, b * b + br * br)
    w, r = 16, 2048
    while r > 64:
        a, b = t[: r // 2], t[r // 2 :]
        u = a + jnp.roll(a, -w, axis=1)
        v = b + jnp.roll(b, w, axis=1)
        t = jnp.where(masks[w], u, v)
        r //= 2
        w //= 2
    ids = idmap_ref[...]
    m = jnp.max(t)
    i = jnp.min(jnp.where(t == m, ids, jnp.int32(8192)))
    row = x_ref[0, pl.ds(i // 2, 1), :]
    row = jnp.where(i % 2 == 1, jnp.roll(row, -64, axis=1), row)
    o_ref[0, 0, :] = row[0, :64]


def kernel(input):
    b, n, d = input.shape
    rows = n * d // 128
    flat = input.reshape(b, rows, 128)
    idmap = jnp.asarray(_IDMAP)
    out = pl.pallas_call(
        _body,
        grid=(b,),
        in_specs=[
            pl.BlockSpec((1, rows, 128), lambda i: (i, 0, 0)),
            pl.BlockSpec((64, 128), lambda i: (0, 0)),
        ],
        out_specs=pl.BlockSpec((1, 1, d), lambda i: (i, 0, 0)),
        out_shape=jax.ShapeDtypeStruct((b, 1, d), input.dtype),
    )(flat, idmap)
    return out.reshape(b, d)
